# Initial kernel scaffold; baseline (speedup 1.0000x reference)
import jax
import jax.numpy as jnp
from jax.experimental import pallas as pl


def kernel(flow_map, event_list, event_mask, dt_input, dt_gt):
    raise NotImplementedError("write your pallas kernel here")


# speedup vs baseline: 1.9048x; 1.9048x over previous
import functools

import jax
import jax.numpy as jnp
from jax import lax
from jax.experimental import pallas as pl
from jax.experimental.pallas import tpu as pltpu
from jax.experimental.pallas import tpu_sc as plsc

_LANES = 16

_GATHER_DNUMS = lax.GatherDimensionNumbers(
    offset_dims=(), collapsed_slice_dims=(0,), start_index_map=(0,))


def _take16(vec, idx):
    return lax.gather(vec, idx[:, None], _GATHER_DNUMS, (1,),
                      mode=lax.GatherScatterMode.PROMISE_IN_BOUNDS)


@functools.lru_cache(maxsize=None)
def _build_gather_kernel(B, N, H, W):
    info = plsc.get_sparse_core_info()
    NC, NS = info.num_cores, info.num_subcores
    NW = NC * NS
    HW = H * W
    TOTAL = B * N
    assert TOTAL % NW == 0
    per_tile = TOTAL // NW

    C = min(12800, per_tile)
    assert C % _LANES == 0
    bases = [j * C for j in range(per_tile // C)]
    if per_tile % C:
        bases.append(per_tile - C)
    assert all(b % 8 == 0 for b in bases)

    mesh = plsc.VectorSubcoreMesh(core_axis_name="c", subcore_axis_name="s")

    @functools.partial(
        pl.kernel,
        mesh=mesh,
        out_type=jax.ShapeDtypeStruct((TOTAL * 2,), jnp.float32),
        scratch_types=[
            pltpu.VMEM((C,), jnp.float32),
            pltpu.VMEM((C,), jnp.float32),
            pltpu.VMEM((2 * C,), jnp.int32),
            pltpu.VMEM((2 * C,), jnp.float32),
            pltpu.SemaphoreType.DMA,
        ],
    )
    def gather_kernel(x_hbm, y_hbm, table_hbm, out_hbm, x_v, y_v, idx_v, rows_v, sem):
        wid = lax.axis_index("s") * NC + lax.axis_index("c")
        tiles_per_batch = NW // B
        batch = wid // tiles_per_batch
        table_off = (batch * (2 * HW)).astype(jnp.float32)
        tile_base = wid * per_tile

        lane = lax.iota(jnp.int32, _LANES)
        dup_lo = lane >> 1
        dup_hi = dup_lo + 8
        parity = lane & 1

        for base in bases:
            evbase = tile_base + base
            pltpu.sync_copy(x_hbm.at[pl.ds(evbase, C)], x_v)
            pltpu.sync_copy(y_hbm.at[pl.ds(evbase, C)], y_v)

            def body(i, carry):
                xv = x_v[pl.ds(i * _LANES, _LANES)]
                yv = y_v[pl.ds(i * _LANES, _LANES)]
                t = (xv + yv * float(W)) * 2.0 + table_off
                ti = t.astype(jnp.int32)
                lo = _take16(ti, dup_lo) + parity
                hi = _take16(ti, dup_hi) + parity
                idx_v[pl.ds(2 * i * _LANES, _LANES)] = lo
                idx_v[pl.ds((2 * i + 1) * _LANES, _LANES)] = hi
                return carry

            lax.fori_loop(0, C // _LANES, body, 0)

            copy = pltpu.async_copy(table_hbm.at[idx_v], rows_v, sem)
            copy.wait()
            pltpu.sync_copy(rows_v, out_hbm.at[pl.ds(2 * evbase, 2 * C)])

    return gather_kernel


def kernel(flow_map, event_list, event_mask, dt_input, dt_gt):
    B, _, H, W = flow_map.shape
    N = event_list.shape[1]
    table = flow_map.reshape(B, 2, H * W).transpose(0, 2, 1).reshape(B * H * W * 2)
    x_col = event_list[:, :, 1].reshape(B * N)
    y_col = event_list[:, :, 2].reshape(B * N)
    out = _build_gather_kernel(B, N, H, W)(x_col, y_col, table)
    return out.reshape(B, N, 2)
